# baseline (speedup 1.0000x reference)
import functools

import jax
import jax.numpy as jnp
from jax import lax
from jax.experimental import pallas as pl
from jax.experimental.pallas import tpu as pltpu
from jax.experimental.pallas import tpu_sc as plsc

_B = 4
_TRACK = 8192
_D = 1024
_LANES = 16
_NSL = _D // _LANES

_NC = 2
_NS = 16
_NW = _NC * _NS

_TPW = _TRACK // _NW
_C = 8
_NCHUNK = _TPW // _C
_NBUF = 3


def _sc_body(x_hbm, t_hbm, o_hbm, x_v, t_v, sem_x, sem_t, sem_o):
    wid = lax.axis_index("s") * _NC + lax.axis_index("c")
    trow0 = wid * _TPW

    def in_copies(g, slot):
        r = trow0 + g * _C
        pltpu.make_async_copy(
            t_hbm.at[pl.ds(r, _C), :], t_v.at[slot], sem_t).start()
        pltpu.make_async_copy(
            x_hbm.at[:, pl.ds(r, _C), :], x_v.at[slot], sem_x).start()

    def wait_in(g, slot):
        r = trow0 + g * _C
        pltpu.make_async_copy(
            t_hbm.at[pl.ds(r, _C), :], t_v.at[slot], sem_t).wait()
        pltpu.make_async_copy(
            x_hbm.at[:, pl.ds(r, _C), :], x_v.at[slot], sem_x).wait()

    def out_copies(g, slot, fn):
        r = trow0 + g * _C
        cp = pltpu.make_async_copy(
            x_v.at[slot], o_hbm.at[:, pl.ds(r, _C), :], sem_o)
        getattr(cp, fn)()

    for g in range(_NBUF - 1):
        in_copies(g, g)

    def chunk_step(g, _):
        slot = g % _NBUF
        wait_in(g, slot)

        @plsc.parallel_loop(0, _C, 1)
        def row_add(r):
            K = 8
            for j0 in range(0, _NSL, K):
                sls = [pl.ds((j0 + k) * _LANES, _LANES) for k in range(K)]
                t16s = [t_v[slot, r, sl] for sl in sls]
                for k in range(K):
                    for b in range(_B):
                        plsc.addupdate(x_v.at[slot, b, r, sls[k]], t16s[k])

        out_copies(g, slot, "start")

        @pl.when(g + _NBUF - 1 < _NCHUNK)
        def _():
            @pl.when(g >= 1)
            def _():
                out_copies(g - 1, (g - 1) % _NBUF, "wait")

            in_copies(g + _NBUF - 1, (g + _NBUF - 1) % _NBUF)

        return 0

    lax.fori_loop(0, _NCHUNK, chunk_step, 0)

    for g in range(_NCHUNK - _NBUF, _NCHUNK):
        out_copies(g, g % _NBUF, "wait")


@jax.jit
def _sc_add(inputs, table):
    mesh = plsc.VectorSubcoreMesh(core_axis_name="c", subcore_axis_name="s")
    fn = functools.partial(
        pl.kernel,
        out_type=jax.ShapeDtypeStruct((_B, _TRACK, _D), jnp.float32),
        mesh=mesh,
        scratch_types=[
            pltpu.VMEM((_NBUF, _B, _C, _D), jnp.float32),
            pltpu.VMEM((_NBUF, _C, _D), jnp.float32),
            pltpu.SemaphoreType.DMA,
            pltpu.SemaphoreType.DMA,
            pltpu.SemaphoreType.DMA,
        ],
    )(_sc_body)
    return fn(inputs, table)


def kernel(inputs, table):
    return _sc_add(inputs, table)
